# baseline (speedup 1.0000x reference)
import functools

import jax
import jax.numpy as jnp
from jax import lax
from jax.experimental import pallas as pl
from jax.experimental.pallas import tpu as pltpu
from jax.experimental.pallas import tpu_sc as plsc

BATCH = 16384
EMBED_DIM = 64
NUM_WORKERS = 32
B_PER_W = BATCH // NUM_WORKERS
GROUPS = B_PER_W // 16
LANE = 128
NCOLS_TOTAL = 7813
CPW = 245
K = 16
STAGED = (BATCH + 1) * EMBED_DIM
NVEC = BATCH // 16


def _lane():
    return lax.broadcasted_iota(jnp.int32, (16,), 0)


def _f16(x):
    return jnp.full((16,), x, jnp.int32)


def _gather_kernel(uidx_hbm, iidx_hbm, user_t, item_t, staged_u, staged_i,
                   idxbuf, hit_u, hit_b, slot_u, slot_b, cb0, cb1, cb2, cb3,
                   rowtmp, counts, misc, fs0, fs1, fs2, fs3, row_sem):
    wid = lax.axis_index("s") * 2 + lax.axis_index("c")
    c0 = wid * CPW
    c1 = jnp.minimum(c0 + CPW, NCOLS_TOTAL)
    ncols = c1 - c0
    lo = c0 * LANE
    hi = c1 * LANE
    lane = _lane()
    lane0 = lane == 0
    cbs = (cb0, cb1, cb2, cb3)
    fsems = (fs0, fs1, fs2, fs3)

    def process_table(table_t, idx_hbm, staged):
        def z_counts(j, c):
            counts[j] = 0
            return c

        lax.fori_loop(0, 256, z_counts, 0)
        misc[0] = 0

        def z_slot(j, c):
            slot_b[pl.ds(j * 16, 16)] = _f16(-1)
            return c

        lax.fori_loop(0, CPW, z_slot, 0)

        pltpu.sync_copy(idx_hbm, idxbuf)

        def scan_body(k, nh):
            v = idxbuf[pl.ds(k * 16, 16)]
            m = (v >= lo) & (v < hi)
            mi = m.astype(jnp.int32)
            pos = nh + jnp.cumsum(mi) - 1
            plsc.store_scatter(hit_u, [pos], v, mask=m)
            plsc.store_scatter(hit_b, [pos], k * 16 + lane, mask=m)
            return nh + jnp.sum(mi)

        nh = lax.fori_loop(0, NVEC, scan_body, 0)

        def place_body(g, c):
            uv = hit_u[pl.ds(g * 16, 16)]
            bv = hit_b[pl.ds(g * 16, 16)]
            for k in range(16):

                @pl.when(g * 16 + k < nh)
                def _():
                    u = uv[k]
                    b = bv[k]
                    cc = (u >> 7) - c0
                    s = counts[cc]

                    @pl.when(s < K)
                    def _():
                        plsc.store_scatter(slot_u, [_f16(cc * K + s)],
                                           _f16(u), mask=lane0)
                        plsc.store_scatter(slot_b, [_f16(cc * K + s)],
                                           _f16(b), mask=lane0)

                    @pl.when(s >= K)
                    def _():
                        nfb = misc[0]
                        plsc.store_scatter(hit_u, [_f16(BATCH - 1 - nfb)],
                                           _f16(u), mask=lane0)
                        plsc.store_scatter(hit_b, [_f16(BATCH - 1 - nfb)],
                                           _f16(b), mask=lane0)
                        misc[0] = nfb + 1

                    counts[cc] = s + 1
            return c

        lax.fori_loop(0, (nh + 15) // 16, place_body, 0)

        def fire_col(j, slot):
            off = pl.multiple_of((c0 + j) * LANE, LANE)
            pltpu.async_copy(
                table_t.at[pl.ds(0, EMBED_DIM), pl.ds(off, LANE)],
                cbs[slot], fsems[slot])

        def wait_col(slot):
            pltpu.make_async_copy(
                table_t.at[pl.ds(0, EMBED_DIM), pl.ds(0, LANE)],
                cbs[slot], fsems[slot]).wait()

        def wait_rows_all():
            pltpu.make_async_copy(
                staged.at[pl.ds(0, 16 * EMBED_DIM)], rowtmp, row_sem).wait()

        def wait_row_one():
            pltpu.make_async_copy(
                staged.at[pl.ds(0, EMBED_DIM)],
                rowtmp.at[pl.ds(0, EMBED_DIM)], row_sem).wait()

        def extract_row(slot, k, u_sel, off_words):
            for c4 in range(4):
                vals = plsc.load_gather(
                    cbs[slot], [c4 * 16 + lane, _f16(u_sel)])
                rowtmp[pl.ds(k * EMBED_DIM + c4 * 16, 16)] = vals
            pltpu.async_copy(
                rowtmp.at[pl.ds(k * EMBED_DIM, EMBED_DIM)],
                staged.at[pl.ds(off_words, EMBED_DIM)], row_sem)

        for p in range(3):
            fire_col(p, p)

        def sweep_body(j4, c):
            for p in range(4):
                j = j4 * 4 + p

                @pl.when(j + 3 < ncols)
                def _():
                    fire_col(j + 3, (p + 3) & 3)

                @pl.when(j < ncols)
                def _():
                    wait_col(p)

                    @pl.when(j > 0)
                    def _():
                        wait_rows_all()

                    uv = slot_u[pl.ds(j * K, 16)]
                    bv = slot_b[pl.ds(j * K, 16)]
                    for k in range(16):
                        b = bv[k]
                        u = uv[k]
                        u_sel = jnp.where(b >= 0, u & 127, 0)
                        off = jnp.where(b >= 0, b, BATCH) * EMBED_DIM
                        extract_row(p, k, u_sel, off)
            return c

        lax.fori_loop(0, (ncols + 3) // 4, sweep_body, 0)
        wait_rows_all()

        nfb = misc[0]

        def fb_u_of(h4, l):
            blk = hit_u[pl.ds(BATCH - 16 - h4 * 16, 16)]
            return blk[l]

        def fb_b_of(h4, l):
            blk = hit_b[pl.ds(BATCH - 16 - h4 * 16, 16)]
            return blk[l]

        def fire_fb(u, slot):
            off = pl.multiple_of((u >> 7) << 7, LANE)
            pltpu.async_copy(
                table_t.at[pl.ds(0, EMBED_DIM), pl.ds(off, LANE)],
                cbs[slot], fsems[slot])

        for p in range(3):

            @pl.when(p < nfb)
            def _():
                fire_fb(fb_u_of(0, 15 - p), p)

        def fb_body(h4, c):
            ublk = hit_u[pl.ds(BATCH - 16 - h4 * 16, 16)]
            bblk = hit_b[pl.ds(BATCH - 16 - h4 * 16, 16)]
            nblk = hit_u[pl.ds(jnp.maximum(BATCH - 32 - h4 * 16, 0), 16)]
            for k in range(16):
                h = h4 * 16 + k
                un = ublk[12 - k] if k <= 12 else nblk[28 - k]

                @pl.when(h + 3 < nfb)
                def _():
                    fire_fb(un, (k + 3) & 3)

                if k == 0:

                    @pl.when((h4 > 0) & (h4 * 16 < nfb))
                    def _():
                        wait_rows_all()

                @pl.when(h < nfb)
                def _():
                    wait_col(k & 3)
                    u = ublk[15 - k]
                    b = bblk[15 - k]
                    extract_row(k & 3, k, u & 127, b * EMBED_DIM)
            return c

        lax.fori_loop(0, (nfb + 15) // 16, fb_body, 0)
        rem = jnp.where(nfb > 0, nfb - ((nfb + 15) // 16 - 1) * 16, 0)

        def fb_drain(r, c):
            wait_row_one()
            return c

        lax.fori_loop(0, rem, fb_drain, 0)

    process_table(user_t, uidx_hbm, staged_u)
    process_table(item_t, iidx_hbm, staged_i)


def _compute_kernel(staged_u, staged_i, wb_hbm, out_hbm, u_rows, i_rows,
                    w_v, out_v):
    wid = lax.axis_index("s") * 2 + lax.axis_index("c")
    base = wid * B_PER_W
    pltpu.sync_copy(staged_u.at[pl.ds(base * EMBED_DIM, B_PER_W * EMBED_DIM)],
                    u_rows)
    pltpu.sync_copy(staged_i.at[pl.ds(base * EMBED_DIM, B_PER_W * EMBED_DIM)],
                    i_rows)
    pltpu.sync_copy(wb_hbm, w_v)

    w_chunks = [w_v[pl.ds(c * 16, 16)] for c in range(5)]
    w_s = [w_chunks[d // 16][d % 16] for d in range(EMBED_DIM)]
    b_s = w_chunks[4][0]
    lane = _lane()

    def group_body(g, carry):
        row_base = (g * 16 + lane) * EMBED_DIM
        acc = jnp.zeros((16,), jnp.float32)
        for d in range(EMBED_DIM):
            u_v = plsc.load_gather(u_rows, [row_base + d])
            i_v = plsc.load_gather(i_rows, [row_base + d])
            acc = acc + u_v * i_v * w_s[d]
        x = acc + b_s
        y = 1.0 / (1.0 + jnp.exp(-x))
        out_v[pl.ds(g * 16, 16)] = y
        return carry

    lax.fori_loop(0, GROUPS, group_body, 0)
    pltpu.sync_copy(out_v, out_hbm.at[pl.ds(base, B_PER_W)])


@jax.jit
def _gmf(user_indices, item_indices, user_table, item_table, W, b):
    mesh = plsc.VectorSubcoreMesh(core_axis_name="c", subcore_axis_name="s")
    params = pltpu.CompilerParams(needs_layout_passes=False)
    kern_a = functools.partial(
        pl.kernel,
        mesh=mesh,
        out_type=(jax.ShapeDtypeStruct((STAGED,), jnp.float32),
                  jax.ShapeDtypeStruct((STAGED,), jnp.float32)),
        scratch_types=[
            pltpu.VMEM((BATCH,), jnp.int32),
            pltpu.VMEM((BATCH,), jnp.int32),
            pltpu.VMEM((BATCH,), jnp.int32),
            pltpu.VMEM((CPW * K,), jnp.int32),
            pltpu.VMEM((CPW * K,), jnp.int32),
            pltpu.VMEM((EMBED_DIM, LANE), jnp.float32),
            pltpu.VMEM((EMBED_DIM, LANE), jnp.float32),
            pltpu.VMEM((EMBED_DIM, LANE), jnp.float32),
            pltpu.VMEM((EMBED_DIM, LANE), jnp.float32),
            pltpu.VMEM((16 * EMBED_DIM,), jnp.float32),
            pltpu.SMEM((256,), jnp.int32),
            pltpu.SMEM((8,), jnp.int32),
            pltpu.SemaphoreType.DMA,
            pltpu.SemaphoreType.DMA,
            pltpu.SemaphoreType.DMA,
            pltpu.SemaphoreType.DMA,
            pltpu.SemaphoreType.DMA,
        ],
        compiler_params=params,
    )(_gather_kernel)
    kern_b = functools.partial(
        pl.kernel,
        mesh=mesh,
        out_type=jax.ShapeDtypeStruct((BATCH,), jnp.float32),
        scratch_types=[
            pltpu.VMEM((B_PER_W * EMBED_DIM,), jnp.float32),
            pltpu.VMEM((B_PER_W * EMBED_DIM,), jnp.float32),
            pltpu.VMEM((EMBED_DIM + 16,), jnp.float32),
            pltpu.VMEM((B_PER_W,), jnp.float32),
        ],
        compiler_params=params,
    )(_compute_kernel)
    uidx = user_indices.astype(jnp.int32)
    iidx = item_indices.astype(jnp.int32)
    wb = jnp.concatenate(
        [W.astype(jnp.float32).reshape(EMBED_DIM),
         jnp.pad(b.astype(jnp.float32), (0, 15))])
    staged_u, staged_i = kern_a(uidx, iidx, user_table.T, item_table.T)
    return kern_b(staged_u, staged_i, wb)


def kernel(user_indices, item_indices, user_table, item_table, W, b):
    return _gmf(user_indices, item_indices, user_table, item_table, W, b)
